# baseline (speedup 1.0000x reference)
import jax
import jax.numpy as jnp
from jax import lax
from jax.experimental import pallas as pl
from jax.experimental.pallas import tpu as pltpu
from jax.experimental.pallas import tpu_sc as plsc

B, H, W, C = 8, 56, 56, 192
HW = H * W
ROWS = B * HW
NW = 32
RPW = ROWS // NW
WPB = HW // RPW
CH = 112
NCH = RPW // CH
NBUF = 4
LANES = 16
CA = 128
CB = C - CA
IDXPAD = -(-RPW // LANES) * LANES


def _body(xf, r, out, idx_v, bufs_a, gsems, bsems, wsems_a):
    wid = lax.axis_index("s") * 2 + lax.axis_index("c")
    b = wid // WPB
    p0 = (wid % WPB) * RPW
    boff = b * HW

    pltpu.sync_copy(r.at[pl.ds(p0, RPW)], idx_v.at[pl.ds(0, RPW)])
    for i in range(IDXPAD // LANES):
        sl = pl.ds(i * LANES, LANES)
        idx_v[sl] = idx_v[sl] + boff

    base = wid * RPW

    def row_group(k, _):
        v = idx_v[pl.ds(k * LANES, LANES)]
        q0 = base + k * LANES
        for jj in range(LANES):
            pltpu.async_copy(
                xf.at[pl.ds(v[jj], 1), pl.ds(CA, CB)],
                out.at[pl.ds(q0 + jj, 1), pl.ds(CA, CB)],
                bsems[0],
            )
        return _

    lax.fori_loop(0, RPW // LANES, row_group, None)

    def start_gather(c):
        s = c % NBUF
        return pltpu.async_copy(
            xf.at[idx_v.at[pl.ds(c * CH, CH)], pl.ds(0, CA)], bufs_a[s], gsems[s]
        )

    def start_write(c):
        s = c % NBUF
        return pltpu.async_copy(
            bufs_a[s], out.at[pl.ds(base + c * CH, CH), pl.ds(0, CA)], wsems_a[s]
        )

    def wait_write(c):
        s = c % NBUF
        pltpu.make_async_copy(
            bufs_a[s], out.at[pl.ds(base + c * CH, CH), pl.ds(0, CA)], wsems_a[s]
        ).wait()

    gathers = [None] * NBUF
    for j in range(min(NBUF, NCH)):
        gathers[j] = start_gather(j)
    written = [None] * NBUF
    for c in range(NCH):
        gathers[c % NBUF].wait()
        start_write(c)
        written[c % NBUF] = c
        n = c + NBUF
        if n < NCH:
            wait_write(c)
            written[c % NBUF] = None
            gathers[n % NBUF] = start_gather(n)
    for c in written:
        if c is not None:
            wait_write(c)

    pltpu.make_async_copy(
        xf.at[pl.ds(0, RPW), pl.ds(CA, CB)],
        out.at[pl.ds(base, RPW), pl.ds(CA, CB)],
        bsems[0],
    ).wait()


@jax.jit
def kernel(x, r):
    xf = x.reshape(ROWS, C)
    r = r.astype(jnp.int32)
    mesh = plsc.VectorSubcoreMesh(core_axis_name="c", subcore_axis_name="s")
    out = pl.kernel(
        _body,
        out_type=jax.ShapeDtypeStruct((ROWS, C), jnp.float32),
        mesh=mesh,
        compiler_params=pltpu.CompilerParams(use_tc_tiling_on_sc=True),
        scratch_types=[
            pltpu.VMEM((IDXPAD,), jnp.int32),
            [pltpu.VMEM((CH, CA), jnp.float32) for _ in range(NBUF)],
            [pltpu.SemaphoreType.DMA for _ in range(NBUF)],
            [pltpu.SemaphoreType.DMA for _ in range(1)],
            [pltpu.SemaphoreType.DMA for _ in range(NBUF)],
        ],
    )(xf, r)
    return out.reshape(B, H, W, C)


# speedup vs baseline: 8.6632x; 8.6632x over previous
import jax
import jax.numpy as jnp
from jax import lax
from jax.experimental import pallas as pl
from jax.experimental.pallas import tpu as pltpu
from jax.experimental.pallas import tpu_sc as plsc

B, H, W, C = 8, 56, 56, 192
HW = H * W
ROWS = B * HW
NW = 32
RPW = ROWS // NW
WPB = HW // RPW
CH = 112
NCH = RPW // CH
NBUF = 3
LANES = 16
CA = 128
CB = C - CA
IDXPAD = -(-RPW // LANES) * LANES


def _body(xf, aux, r, out, idx_v, bufs_a, bufs_b, bufs_c,
          gsems, bsems, wsems_a, wsems_b):
    wid = lax.axis_index("s") * 2 + lax.axis_index("c")
    b = wid // WPB
    p0 = (wid % WPB) * RPW
    boff = b * HW

    pltpu.sync_copy(r.at[pl.ds(p0, RPW)], idx_v.at[pl.ds(0, RPW)])
    for i in range(IDXPAD // LANES):
        sl = pl.ds(i * LANES, LANES)
        idx_v[sl] = idx_v[sl] + boff

    base = wid * RPW

    def start_gather(c):
        s = c % NBUF
        idx = idx_v.at[pl.ds(c * CH, CH)]
        ga = pltpu.async_copy(xf.at[idx, pl.ds(0, CA)], bufs_a[s], gsems[s])
        pltpu.async_copy(aux.at[idx], bufs_b[s], bsems[s])
        return ga

    def wait_gather_b(c):
        s = c % NBUF
        pltpu.make_async_copy(
            aux.at[idx_v.at[pl.ds(c * CH, CH)]], bufs_b[s], bsems[s]
        ).wait()

    def compact_b(c):
        s = c % NBUF

        def move_row(j, _):
            for k in range(CB // LANES):
                bufs_c[s][j, pl.ds(k * LANES, LANES)] = (
                    bufs_b[s][j, pl.ds(k * LANES, LANES)]
                )
            return _

        lax.fori_loop(0, CH, move_row, None)

    def start_write(c):
        s = c % NBUF
        rows = pl.ds(base + c * CH, CH)
        pltpu.async_copy(bufs_a[s], out.at[rows, pl.ds(0, CA)], wsems_a[s])
        pltpu.async_copy(
            bufs_c[s], out.at[rows, pl.ds(CA, CB)], wsems_b[s]
        )

    def wait_write(c):
        s = c % NBUF
        rows = pl.ds(base + c * CH, CH)
        pltpu.make_async_copy(
            bufs_a[s], out.at[rows, pl.ds(0, CA)], wsems_a[s]
        ).wait()
        pltpu.make_async_copy(
            bufs_c[s], out.at[rows, pl.ds(CA, CB)], wsems_b[s]
        ).wait()

    gathers = [None] * NBUF
    for j in range(min(NBUF, NCH)):
        gathers[j] = start_gather(j)
    written = [None] * NBUF
    for c in range(NCH):
        gathers[c % NBUF].wait()
        wait_gather_b(c)
        compact_b(c)
        start_write(c)
        written[c % NBUF] = c
        n = c + NBUF
        if n < NCH:
            wait_write(c)
            written[c % NBUF] = None
            gathers[n % NBUF] = start_gather(n)
    for c in written:
        if c is not None:
            wait_write(c)


@jax.jit
def kernel(x, r):
    xf = x.reshape(ROWS, C)
    r = r.astype(jnp.int32)
    aux = jnp.concatenate(
        [xf[:, CA:], jnp.zeros((ROWS, CA - CB), jnp.float32)], axis=1
    )
    mesh = plsc.VectorSubcoreMesh(core_axis_name="c", subcore_axis_name="s")
    out = pl.kernel(
        _body,
        out_type=jax.ShapeDtypeStruct((ROWS, C), jnp.float32),
        mesh=mesh,
        compiler_params=pltpu.CompilerParams(use_tc_tiling_on_sc=True),
        scratch_types=[
            pltpu.VMEM((IDXPAD,), jnp.int32),
            [pltpu.VMEM((CH, CA), jnp.float32) for _ in range(NBUF)],
            [pltpu.VMEM((CH, CA), jnp.float32) for _ in range(NBUF)],
            [pltpu.VMEM((CH, CB), jnp.float32) for _ in range(NBUF)],
            [pltpu.SemaphoreType.DMA for _ in range(NBUF)],
            [pltpu.SemaphoreType.DMA for _ in range(NBUF)],
            [pltpu.SemaphoreType.DMA for _ in range(NBUF)],
            [pltpu.SemaphoreType.DMA for _ in range(NBUF)],
        ],
    )(xf, aux, r)
    return out.reshape(B, H, W, C)
